# Initial kernel scaffold; baseline (speedup 1.0000x reference)
import jax
import jax.numpy as jnp
from jax.experimental import pallas as pl


def kernel(x, Wg, bg, W1, b1, W2, b2):
    raise NotImplementedError("write your pallas kernel here")


# speedup vs baseline: 10.8274x; 10.8274x over previous
import functools

import jax
import jax.numpy as jnp
from jax import lax
from jax.experimental import pallas as pl
from jax.experimental.pallas import tpu as pltpu

K = 2
BLK = 128


def _ffn_tile(te_ref, xs_ref, w_ref, W1_ref, b1_ref, W2_ref, b2_ref, out_ref):
    i = pl.program_id(0)
    nt = pl.num_programs(0)

    @pl.when(i < te_ref[nt])
    def _():
        xs = xs_ref[...]
        h = lax.dot_general(
            xs, W1_ref[0], (((1,), (1,)), ((), ())),
            preferred_element_type=jnp.float32,
        ) + b1_ref[0]
        h = 0.5 * h * (1.0 + lax.erf(h * 0.7071067811865476))
        ys = lax.dot_general(
            h, W2_ref[0], (((1,), (1,)), ((), ())),
            preferred_element_type=jnp.float32,
        ) + b2_ref[0]
        out_ref[...] = ys * w_ref[...]


def kernel(x, Wg, bg, W1, b1, W2, b2):
    Bq, Sq, D = x.shape
    E, F, _ = W1.shape
    T = Bq * Sq
    flat = x.reshape(T, D)

    scores = flat @ Wg.T + bg
    topv, topi = lax.top_k(scores, K)
    gates = jax.nn.softmax(topv, axis=-1)

    e_flat = topi.reshape(-1)
    oh = jax.nn.one_hot(e_flat, E, dtype=jnp.int32)
    cum = jnp.cumsum(oh, axis=0) - oh
    rank = jnp.take_along_axis(cum, e_flat[:, None], axis=1)[:, 0]
    counts = jnp.sum(oh, axis=0)

    tiles_per_e = (counts + BLK - 1) // BLK
    tile_cum = jnp.cumsum(tiles_per_e)
    num_active = tile_cum[-1]
    NT = T * K // BLK + E
    PADDED = NT * BLK
    pad_off = (tile_cum - tiles_per_e) * BLK

    tile_ids = jnp.arange(NT, dtype=jnp.int32)
    tile_expert = jnp.searchsorted(tile_cum, tile_ids, side="right").astype(jnp.int32)
    last_e = jnp.argmax(jnp.where(counts > 0, jnp.arange(E), -1)).astype(jnp.int32)
    tile_expert = jnp.where(tile_ids < num_active, tile_expert, last_e)
    te = jnp.concatenate([tile_expert, num_active[None].astype(jnp.int32)])

    ppos = pad_off[e_flat] + rank

    tok = jnp.arange(T * K, dtype=jnp.int32) // K
    xs = jnp.zeros((PADDED, D), flat.dtype).at[ppos].set(flat[tok])
    w_pad = jnp.zeros((PADDED, 1), flat.dtype).at[ppos, 0].set(gates.reshape(-1))

    grid_spec = pltpu.PrefetchScalarGridSpec(
        num_scalar_prefetch=1,
        grid=(NT,),
        in_specs=[
            pl.BlockSpec((BLK, D), lambda i, te: (i, 0)),
            pl.BlockSpec((BLK, 1), lambda i, te: (i, 0)),
            pl.BlockSpec((1, F, D), lambda i, te: (te[i], 0, 0)),
            pl.BlockSpec((1, 1, F), lambda i, te: (te[i], 0, 0)),
            pl.BlockSpec((1, D, F), lambda i, te: (te[i], 0, 0)),
            pl.BlockSpec((1, 1, D), lambda i, te: (te[i], 0, 0)),
        ],
        out_specs=pl.BlockSpec((BLK, D), lambda i, te: (i, 0)),
    )
    ys = pl.pallas_call(
        _ffn_tile,
        grid_spec=grid_spec,
        out_shape=jax.ShapeDtypeStruct((PADDED, D), jnp.float32),
    )(te, xs, w_pad, W1, b1.reshape(E, 1, F), W2, b2.reshape(E, 1, D))

    out = jnp.sum(ys[ppos].reshape(T, K, D), axis=1)
    return out.reshape(Bq, Sq, D)


# speedup vs baseline: 15.8174x; 1.4609x over previous
import functools

import jax
import jax.numpy as jnp
from jax import lax
from jax.experimental import pallas as pl
from jax.experimental.pallas import tpu as pltpu
from jax.experimental.pallas import tpu_sc as plsc

K = 2
BLK = 128
GTILE = 128
WREP = 128


def _gating_tile(x_ref, wg_ref, bg_ref, topi_ref, rank_ref, g0_ref, g1_ref,
                 counts_ref, counter):
    i = pl.program_id(0)

    @pl.when(i == 0)
    def _():
        counter[...] = jnp.zeros_like(counter)

    E = wg_ref.shape[0]
    xs = x_ref[...]
    scores = lax.dot_general(
        xs, wg_ref[...], (((1,), (1,)), ((), ())),
        preferred_element_type=jnp.float32,
    ) + bg_ref[...]

    col = lax.broadcasted_iota(jnp.int32, (GTILE, E), 1)
    m1 = jnp.max(scores, axis=1, keepdims=True)
    a1 = jnp.argmax(scores, axis=1).astype(jnp.int32)
    oh1 = col == a1[:, None]
    masked = jnp.where(oh1, -jnp.inf, scores)
    m2 = jnp.max(masked, axis=1, keepdims=True)
    a2 = jnp.argmax(masked, axis=1).astype(jnp.int32)
    oh2 = col == a2[:, None]

    t = jnp.exp(m2 - m1)
    g1 = 1.0 / (1.0 + t)
    g2 = 1.0 - g1

    occ = oh1.astype(jnp.float32) + oh2.astype(jnp.float32)
    ri = lax.broadcasted_iota(jnp.int32, (GTILE, GTILE), 0)
    ci = lax.broadcasted_iota(jnp.int32, (GTILE, GTILE), 1)
    ltri = (ci < ri).astype(jnp.float32)
    cumexc = lax.dot_general(
        ltri, occ, (((1,), (0,)), ((), ())),
        preferred_element_type=jnp.float32,
    ) + counter[...]
    r0 = jnp.sum(jnp.where(oh1, cumexc, 0.0), axis=1)
    r1 = jnp.sum(jnp.where(oh2, cumexc, 0.0), axis=1)

    counter[...] = counter[...] + jnp.sum(occ, axis=0, keepdims=True)
    counts_ref[...] = counter[...]
    topi_ref[...] = jnp.concatenate([a1[:, None], a2[:, None]], axis=1)
    rank_ref[...] = jnp.concatenate(
        [r0[:, None], r1[:, None]], axis=1).astype(jnp.int32)
    g0_ref[...] = jnp.broadcast_to(g1, (GTILE, WREP))
    g1_ref[...] = jnp.broadcast_to(g2, (GTILE, WREP))


def _posmap_tile(topi_ref, rank_ref, off_ref, p0_ref, p1_ref):
    E = off_ref.shape[1]
    col = lax.broadcasted_iota(jnp.int32, (GTILE, E), 1)
    off = off_ref[...]
    a1 = topi_ref[:, 0][:, None]
    a2 = topi_ref[:, 1][:, None]
    o1 = jnp.sum(jnp.where(col == a1, off, 0.0), axis=1)
    o2 = jnp.sum(jnp.where(col == a2, off, 0.0), axis=1)
    p0_ref[...] = o1.astype(jnp.int32)[:, None] + rank_ref[:, 0][:, None]
    p1_ref[...] = o2.astype(jnp.int32)[:, None] + rank_ref[:, 1][:, None]


def _ffn_tile(te_ref, xs_ref, w_ref, W1_ref, b1_ref, W2_ref, b2_ref, out_ref):
    i = pl.program_id(0)
    nt = pl.num_programs(0)

    @pl.when(i < te_ref[nt])
    def _():
        xs = xs_ref[...]
        h = lax.dot_general(
            xs, W1_ref[0], (((1,), (1,)), ((), ())),
            preferred_element_type=jnp.float32,
        ) + b1_ref[0]
        h = 0.5 * h * (1.0 + lax.erf(h * 0.7071067811865476))
        ys = lax.dot_general(
            h, W2_ref[0], (((1,), (1,)), ((), ())),
            preferred_element_type=jnp.float32,
        ) + b2_ref[0]
        out_ref[...] = ys * w_ref[:, :1]


def kernel(x, Wg, bg, W1, b1, W2, b2):
    Bq, Sq, D = x.shape
    E, F, _ = W1.shape
    T = Bq * Sq
    A = T * K
    flat = x.reshape(T, D)

    topi, rank, g0w, g1w, counts_f = pl.pallas_call(
        _gating_tile,
        grid=(T // GTILE,),
        in_specs=[
            pl.BlockSpec((GTILE, D), lambda i: (i, 0)),
            pl.BlockSpec((E, D), lambda i: (0, 0)),
            pl.BlockSpec((1, E), lambda i: (0, 0)),
        ],
        out_specs=[
            pl.BlockSpec((GTILE, K), lambda i: (i, 0)),
            pl.BlockSpec((GTILE, K), lambda i: (i, 0)),
            pl.BlockSpec((GTILE, WREP), lambda i: (i, 0)),
            pl.BlockSpec((GTILE, WREP), lambda i: (i, 0)),
            pl.BlockSpec((1, E), lambda i: (0, 0)),
        ],
        out_shape=[
            jax.ShapeDtypeStruct((T, K), jnp.int32),
            jax.ShapeDtypeStruct((T, K), jnp.int32),
            jax.ShapeDtypeStruct((T, WREP), jnp.float32),
            jax.ShapeDtypeStruct((T, WREP), jnp.float32),
            jax.ShapeDtypeStruct((1, E), jnp.float32),
        ],
        scratch_shapes=[pltpu.VMEM((1, E), jnp.float32)],
    )(flat, Wg, bg.reshape(1, E))

    counts = counts_f[0].astype(jnp.int32)
    tiles_per_e = (counts + BLK - 1) // BLK
    tile_cum = jnp.cumsum(tiles_per_e)
    num_active = tile_cum[-1]
    NT = A // BLK + E
    PADDED = NT * BLK
    pad_off = ((tile_cum - tiles_per_e) * BLK).astype(jnp.float32)

    tile_ids = jnp.arange(NT, dtype=jnp.int32)
    tile_expert = jnp.searchsorted(tile_cum, tile_ids, side="right").astype(jnp.int32)
    last_e = jnp.argmax(jnp.where(counts > 0, jnp.arange(E), -1)).astype(jnp.int32)
    tile_expert = jnp.where(tile_ids < num_active, tile_expert, last_e)
    te = jnp.concatenate([tile_expert, num_active[None].astype(jnp.int32)])

    p0, p1 = pl.pallas_call(
        _posmap_tile,
        grid=(T // GTILE,),
        in_specs=[
            pl.BlockSpec((GTILE, K), lambda i: (i, 0)),
            pl.BlockSpec((GTILE, K), lambda i: (i, 0)),
            pl.BlockSpec((1, E), lambda i: (0, 0)),
        ],
        out_specs=[
            pl.BlockSpec((GTILE, 1), lambda i: (i, 0)),
            pl.BlockSpec((GTILE, 1), lambda i: (i, 0)),
        ],
        out_shape=[
            jax.ShapeDtypeStruct((T, 1), jnp.int32),
            jax.ShapeDtypeStruct((T, 1), jnp.int32),
        ],
    )(topi, rank, pad_off.reshape(1, E))
    p0 = p0.reshape(T)
    p1 = p1.reshape(T)

    info = plsc.get_sparse_core_info()
    NC, NS = info.num_cores, info.num_subcores
    NW = NC * NS
    mesh = plsc.VectorSubcoreMesh(core_axis_name="c", subcore_axis_name="s")

    tok_per_w = T // NW
    SUBT = 64
    NSUB = tok_per_w // SUBT

    @functools.partial(
        pl.kernel,
        out_type=(
            jax.ShapeDtypeStruct((PADDED, D), jnp.float32),
            jax.ShapeDtypeStruct((PADDED, WREP), jnp.float32),
        ),
        mesh=mesh,
        scratch_types=[
            pltpu.VMEM((NSUB, SUBT), jnp.int32),
            pltpu.VMEM((NSUB, SUBT), jnp.int32),
            pltpu.VMEM((NSUB, SUBT), jnp.int32),
            pltpu.VMEM((SUBT, D), jnp.float32),
            pltpu.VMEM((SUBT, WREP), jnp.float32),
            pltpu.SemaphoreType.DMA,
        ],
    )
    def _dispatch(flat_hbm, p0_hbm, p1_hbm, g0_hbm, g1_hbm, xs_hbm, w2_hbm,
                  tok_v, p0_v, p1_v, rows_v, wrow_v, sem):
        wid = lax.axis_index("s") * NC + lax.axis_index("c")
        tbase = wid * tok_per_w
        L = 16
        for j in range(NSUB):
            pltpu.sync_copy(p0_hbm.at[pl.ds(tbase + j * SUBT, SUBT)], p0_v.at[j])
            pltpu.sync_copy(p1_hbm.at[pl.ds(tbase + j * SUBT, SUBT)], p1_v.at[j])
            for v in range(SUBT // L):
                tok_v[j, pl.ds(v * L, L)] = (
                    lax.broadcasted_iota(jnp.int32, (L,), 0)
                    + (tbase + j * SUBT + v * L))
        for j in range(NSUB):
            pltpu.async_copy(flat_hbm.at[tok_v.at[j]], rows_v, sem).wait()
            pltpu.async_copy(rows_v, xs_hbm.at[p0_v.at[j]], sem).wait()
            pltpu.async_copy(rows_v, xs_hbm.at[p1_v.at[j]], sem).wait()
            pltpu.sync_copy(g0_hbm.at[pl.ds(tbase + j * SUBT, SUBT)], wrow_v)
            pltpu.async_copy(wrow_v, w2_hbm.at[p0_v.at[j]], sem).wait()
            pltpu.sync_copy(g1_hbm.at[pl.ds(tbase + j * SUBT, SUBT)], wrow_v)
            pltpu.async_copy(wrow_v, w2_hbm.at[p1_v.at[j]], sem).wait()

    xs, w2 = _dispatch(flat, p0, p1, g0w, g1w)

    grid_spec = pltpu.PrefetchScalarGridSpec(
        num_scalar_prefetch=1,
        grid=(NT,),
        in_specs=[
            pl.BlockSpec((BLK, D), lambda i, te: (i, 0)),
            pl.BlockSpec((BLK, WREP), lambda i, te: (i, 0)),
            pl.BlockSpec((1, F, D), lambda i, te: (te[i], 0, 0)),
            pl.BlockSpec((1, 1, F), lambda i, te: (te[i], 0, 0)),
            pl.BlockSpec((1, D, F), lambda i, te: (te[i], 0, 0)),
            pl.BlockSpec((1, 1, D), lambda i, te: (te[i], 0, 0)),
        ],
        out_specs=pl.BlockSpec((BLK, D), lambda i, te: (i, 0)),
    )
    ys = pl.pallas_call(
        _ffn_tile,
        grid_spec=grid_spec,
        out_shape=jax.ShapeDtypeStruct((PADDED, D), jnp.float32),
    )(te, xs, w2, W1, b1.reshape(E, 1, F), W2, b2.reshape(E, 1, D))

    SUBC = 32
    NSUBC = tok_per_w // SUBC

    @functools.partial(
        pl.kernel,
        out_type=jax.ShapeDtypeStruct((T, D), jnp.float32),
        mesh=mesh,
        scratch_types=[
            pltpu.VMEM((NSUBC, SUBC), jnp.int32),
            pltpu.VMEM((NSUBC, SUBC), jnp.int32),
            pltpu.VMEM((SUBC, D), jnp.float32),
            pltpu.VMEM((SUBC, D), jnp.float32),
            pltpu.SemaphoreType.DMA,
        ],
    )
    def _combine(ys_hbm, p0_hbm, p1_hbm, out_hbm, p0_v, p1_v, rows0_v,
                 rows1_v, sem):
        wid = lax.axis_index("s") * NC + lax.axis_index("c")
        tbase = wid * tok_per_w
        L = 16
        for j in range(NSUBC):
            pltpu.sync_copy(p0_hbm.at[pl.ds(tbase + j * SUBC, SUBC)], p0_v.at[j])
            pltpu.sync_copy(p1_hbm.at[pl.ds(tbase + j * SUBC, SUBC)], p1_v.at[j])
        for j in range(NSUBC):
            pltpu.async_copy(ys_hbm.at[p0_v.at[j]], rows0_v, sem).wait()
            pltpu.async_copy(ys_hbm.at[p1_v.at[j]], rows1_v, sem).wait()

            def body(r, _):
                for c in range(D // L):
                    rows0_v[r, pl.ds(c * L, L)] = (
                        rows0_v[r, pl.ds(c * L, L)]
                        + rows1_v[r, pl.ds(c * L, L)])
                return 0

            lax.fori_loop(0, SUBC, body, 0)
            pltpu.sync_copy(rows0_v, out_hbm.at[pl.ds(tbase + j * SUBC, SUBC)])

    out = _combine(ys, p0, p1)
    return out.reshape(Bq, Sq, D)


# speedup vs baseline: 19.4639x; 1.2305x over previous
import functools

import jax
import jax.numpy as jnp
from jax import lax
from jax.experimental import pallas as pl
from jax.experimental.pallas import tpu as pltpu
from jax.experimental.pallas import tpu_sc as plsc

K = 2
BLK = 256
GTILE = 128
WREP = 128


def _gating_tile(x_ref, wg_ref, bg_ref, topi_ref, rank_ref, g0_ref, g1_ref,
                 counts_ref, counter):
    i = pl.program_id(0)

    @pl.when(i == 0)
    def _():
        counter[...] = jnp.zeros_like(counter)

    E = wg_ref.shape[0]
    xs = x_ref[...]
    scores = lax.dot_general(
        xs, wg_ref[...], (((1,), (1,)), ((), ())),
        preferred_element_type=jnp.float32,
    ) + bg_ref[...]

    col = lax.broadcasted_iota(jnp.int32, (GTILE, E), 1)
    m1 = jnp.max(scores, axis=1, keepdims=True)
    a1 = jnp.argmax(scores, axis=1).astype(jnp.int32)
    oh1 = col == a1[:, None]
    masked = jnp.where(oh1, -jnp.inf, scores)
    m2 = jnp.max(masked, axis=1, keepdims=True)
    a2 = jnp.argmax(masked, axis=1).astype(jnp.int32)
    oh2 = col == a2[:, None]

    t = jnp.exp(m2 - m1)
    g1 = 1.0 / (1.0 + t)
    g2 = 1.0 - g1

    occ = oh1.astype(jnp.float32) + oh2.astype(jnp.float32)
    ri = lax.broadcasted_iota(jnp.int32, (GTILE, GTILE), 0)
    ci = lax.broadcasted_iota(jnp.int32, (GTILE, GTILE), 1)
    ltri = (ci < ri).astype(jnp.float32)
    cumexc = lax.dot_general(
        ltri, occ, (((1,), (0,)), ((), ())),
        preferred_element_type=jnp.float32,
    ) + counter[...]
    r0 = jnp.sum(jnp.where(oh1, cumexc, 0.0), axis=1)
    r1 = jnp.sum(jnp.where(oh2, cumexc, 0.0), axis=1)

    counter[...] = counter[...] + jnp.sum(occ, axis=0, keepdims=True)
    counts_ref[...] = counter[...]
    topi_ref[...] = jnp.concatenate([a1[:, None], a2[:, None]], axis=1)
    rank_ref[...] = jnp.concatenate(
        [r0[:, None], r1[:, None]], axis=1).astype(jnp.int32)
    g0_ref[...] = jnp.broadcast_to(g1, (GTILE, WREP))
    g1_ref[...] = jnp.broadcast_to(g2, (GTILE, WREP))


def _make_posmap(NT):
    def _posmap_tile(topi_ref, rank_ref, counts_ref, p0_ref, p1_ref, te_ref):
        E = counts_ref.shape[1]
        c = counts_ref[...]
        tiles = jnp.floor((c + (BLK - 1)) / BLK)
        ei = lax.broadcasted_iota(jnp.int32, (E, E), 0)
        ej = lax.broadcasted_iota(jnp.int32, (E, E), 1)
        cummat = (ei <= ej).astype(jnp.float32)
        tile_cum = lax.dot_general(
            tiles, cummat, (((1,), (0,)), ((), ())),
            preferred_element_type=jnp.float32,
        )
        pad_off = (tile_cum - tiles) * BLK

        col = lax.broadcasted_iota(jnp.int32, (GTILE, E), 1)
        a1 = topi_ref[:, 0][:, None]
        a2 = topi_ref[:, 1][:, None]
        o1 = jnp.sum(jnp.where(col == a1, pad_off, 0.0), axis=1)
        o2 = jnp.sum(jnp.where(col == a2, pad_off, 0.0), axis=1)
        p0_ref[...] = o1.astype(jnp.int32)[:, None] + rank_ref[:, 0][:, None]
        p1_ref[...] = o2.astype(jnp.int32)[:, None] + rank_ref[:, 1][:, None]

        @pl.when(pl.program_id(0) == 0)
        def _():
            num_active = tile_cum[0, E - 1].astype(jnp.int32)
            ids = lax.broadcasted_iota(jnp.int32, (NT, E), 0).astype(jnp.float32)
            cums = jnp.broadcast_to(tile_cum, (NT, E))
            tev = jnp.sum((cums <= ids).astype(jnp.float32), axis=1)
            tev = tev.astype(jnp.int32)
            eids = lax.broadcasted_iota(jnp.int32, (1, E), 1)
            last_e = jnp.max(
                jnp.where(c > 0, eids, -1), axis=1)[0].astype(jnp.int32)
            tid = lax.broadcasted_iota(jnp.int32, (NT,), 0)
            tev = jnp.where(tid < num_active, tev, last_e)
            te_ref[...] = jnp.concatenate(
                [tev, num_active[None]])[None, :]

    return _posmap_tile


def _ffn_tile(te_ref, xs_ref, w_ref, W1_ref, b1_ref, W2_ref, b2_ref, out_ref):
    i = pl.program_id(0)
    nt = pl.num_programs(0)

    @pl.when(i < te_ref[nt])
    def _():
        xs = xs_ref[...]
        h = lax.dot_general(
            xs, W1_ref[0], (((1,), (1,)), ((), ())),
            preferred_element_type=jnp.float32,
        ) + b1_ref[0]
        h = 0.5 * h * (1.0 + lax.erf(h * 0.7071067811865476))
        ys = lax.dot_general(
            h, W2_ref[0], (((1,), (1,)), ((), ())),
            preferred_element_type=jnp.float32,
        ) + b2_ref[0]
        out_ref[...] = ys * w_ref[:, :1]


def kernel(x, Wg, bg, W1, b1, W2, b2):
    Bq, Sq, D = x.shape
    E, F, _ = W1.shape
    T = Bq * Sq
    A = T * K
    flat = x.reshape(T, D)
    NT = A // BLK + E
    PADDED = NT * BLK

    topi, rank, g0w, g1w, counts_f = pl.pallas_call(
        _gating_tile,
        grid=(T // GTILE,),
        in_specs=[
            pl.BlockSpec((GTILE, D), lambda i: (i, 0)),
            pl.BlockSpec((E, D), lambda i: (0, 0)),
            pl.BlockSpec((1, E), lambda i: (0, 0)),
        ],
        out_specs=[
            pl.BlockSpec((GTILE, K), lambda i: (i, 0)),
            pl.BlockSpec((GTILE, K), lambda i: (i, 0)),
            pl.BlockSpec((GTILE, WREP), lambda i: (i, 0)),
            pl.BlockSpec((GTILE, WREP), lambda i: (i, 0)),
            pl.BlockSpec((1, E), lambda i: (0, 0)),
        ],
        out_shape=[
            jax.ShapeDtypeStruct((T, K), jnp.int32),
            jax.ShapeDtypeStruct((T, K), jnp.int32),
            jax.ShapeDtypeStruct((T, WREP), jnp.float32),
            jax.ShapeDtypeStruct((T, WREP), jnp.float32),
            jax.ShapeDtypeStruct((1, E), jnp.float32),
        ],
        scratch_shapes=[pltpu.VMEM((1, E), jnp.float32)],
    )(flat, Wg, bg.reshape(1, E))

    p0, p1, te2 = pl.pallas_call(
        _make_posmap(NT),
        grid=(T // GTILE,),
        in_specs=[
            pl.BlockSpec((GTILE, K), lambda i: (i, 0)),
            pl.BlockSpec((GTILE, K), lambda i: (i, 0)),
            pl.BlockSpec((1, E), lambda i: (0, 0)),
        ],
        out_specs=[
            pl.BlockSpec((GTILE, 1), lambda i: (i, 0)),
            pl.BlockSpec((GTILE, 1), lambda i: (i, 0)),
            pl.BlockSpec((1, NT + 1), lambda i: (0, 0)),
        ],
        out_shape=[
            jax.ShapeDtypeStruct((T, 1), jnp.int32),
            jax.ShapeDtypeStruct((T, 1), jnp.int32),
            jax.ShapeDtypeStruct((1, NT + 1), jnp.int32),
        ],
    )(topi, rank, counts_f)
    p0 = p0.reshape(T)
    p1 = p1.reshape(T)
    te = te2.reshape(NT + 1)

    info = plsc.get_sparse_core_info()
    NC, NS = info.num_cores, info.num_subcores
    NW = NC * NS
    mesh = plsc.VectorSubcoreMesh(core_axis_name="c", subcore_axis_name="s")
    tok_per_w = T // NW

    SUBT = 32
    NSUB = tok_per_w // SUBT

    @functools.partial(
        pl.kernel,
        out_type=(
            jax.ShapeDtypeStruct((PADDED, D), jnp.float32),
            jax.ShapeDtypeStruct((PADDED, WREP), jnp.float32),
        ),
        mesh=mesh,
        scratch_types=[
            pltpu.VMEM((NSUB, SUBT), jnp.int32),
            pltpu.VMEM((NSUB, SUBT), jnp.int32),
            pltpu.VMEM((NSUB, SUBT), jnp.int32),
            pltpu.VMEM((2, SUBT, D), jnp.float32),
            pltpu.VMEM((2, SUBT, WREP), jnp.float32),
            pltpu.VMEM((2, SUBT, WREP), jnp.float32),
            pltpu.SemaphoreType.DMA,
            pltpu.SemaphoreType.DMA,
        ],
    )
    def _dispatch(flat_hbm, p0_hbm, p1_hbm, g0_hbm, g1_hbm, xs_hbm, w2_hbm,
                  tok_v, p0_v, p1_v, rows_v, w0_v, w1_v, sem_in, sem_out):
        wid = lax.axis_index("s") * NC + lax.axis_index("c")
        tbase = wid * tok_per_w
        L = 16
        for j in range(NSUB):
            pltpu.sync_copy(p0_hbm.at[pl.ds(tbase + j * SUBT, SUBT)], p0_v.at[j])
            pltpu.sync_copy(p1_hbm.at[pl.ds(tbase + j * SUBT, SUBT)], p1_v.at[j])
            for v in range(SUBT // L):
                tok_v[j, pl.ds(v * L, L)] = (
                    lax.broadcasted_iota(jnp.int32, (L,), 0)
                    + (tbase + j * SUBT + v * L))

        def issue_in(j):
            b = j % 2
            return (
                pltpu.async_copy(flat_hbm.at[tok_v.at[j]], rows_v.at[b], sem_in),
                pltpu.async_copy(
                    g0_hbm.at[pl.ds(tbase + j * SUBT, SUBT)], w0_v.at[b], sem_in),
                pltpu.async_copy(
                    g1_hbm.at[pl.ds(tbase + j * SUBT, SUBT)], w1_v.at[b], sem_in),
            )

        def issue_out(j):
            b = j % 2
            return (
                pltpu.async_copy(rows_v.at[b], xs_hbm.at[p0_v.at[j]], sem_out),
                pltpu.async_copy(rows_v.at[b], xs_hbm.at[p1_v.at[j]], sem_out),
                pltpu.async_copy(w0_v.at[b], w2_hbm.at[p0_v.at[j]], sem_out),
                pltpu.async_copy(w1_v.at[b], w2_hbm.at[p1_v.at[j]], sem_out),
            )

        pend_in = issue_in(0)
        pend_out = None
        for j in range(NSUB):
            for d in pend_in:
                d.wait()
            if j + 1 < NSUB:
                if pend_out is not None:
                    for d in pend_out:
                        d.wait()
                pend_in = issue_in(j + 1)
            new_out = issue_out(j)
            if j + 1 >= NSUB and pend_out is not None:
                for d in pend_out:
                    d.wait()
            pend_out = new_out
        for d in pend_out:
            d.wait()

    xs, w2 = _dispatch(flat, p0, p1, g0w, g1w)

    grid_spec = pltpu.PrefetchScalarGridSpec(
        num_scalar_prefetch=1,
        grid=(NT,),
        in_specs=[
            pl.BlockSpec((BLK, D), lambda i, te: (i, 0)),
            pl.BlockSpec((BLK, WREP), lambda i, te: (i, 0)),
            pl.BlockSpec((1, F, D), lambda i, te: (te[i], 0, 0)),
            pl.BlockSpec((1, 1, F), lambda i, te: (te[i], 0, 0)),
            pl.BlockSpec((1, D, F), lambda i, te: (te[i], 0, 0)),
            pl.BlockSpec((1, 1, D), lambda i, te: (te[i], 0, 0)),
        ],
        out_specs=pl.BlockSpec((BLK, D), lambda i, te: (i, 0)),
    )
    ys = pl.pallas_call(
        _ffn_tile,
        grid_spec=grid_spec,
        out_shape=jax.ShapeDtypeStruct((PADDED, D), jnp.float32),
    )(te, xs, w2, W1, b1.reshape(E, 1, F), W2, b2.reshape(E, 1, D))

    SUBC = 32
    NSUBC = tok_per_w // SUBC

    @functools.partial(
        pl.kernel,
        out_type=jax.ShapeDtypeStruct((T, D), jnp.float32),
        mesh=mesh,
        scratch_types=[
            pltpu.VMEM((NSUBC, SUBC), jnp.int32),
            pltpu.VMEM((NSUBC, SUBC), jnp.int32),
            pltpu.VMEM((2, SUBC, D), jnp.float32),
            pltpu.VMEM((2, SUBC, D), jnp.float32),
            pltpu.SemaphoreType.DMA,
            pltpu.SemaphoreType.DMA,
        ],
    )
    def _combine(ys_hbm, p0_hbm, p1_hbm, out_hbm, p0_v, p1_v, rows0_v,
                 rows1_v, sem_in, sem_out):
        wid = lax.axis_index("s") * NC + lax.axis_index("c")
        tbase = wid * tok_per_w
        L = 16
        for j in range(NSUBC):
            pltpu.sync_copy(p0_hbm.at[pl.ds(tbase + j * SUBC, SUBC)], p0_v.at[j])
            pltpu.sync_copy(p1_hbm.at[pl.ds(tbase + j * SUBC, SUBC)], p1_v.at[j])

        def issue_in(j):
            b = j % 2
            return (
                pltpu.async_copy(ys_hbm.at[p0_v.at[j]], rows0_v.at[b], sem_in),
                pltpu.async_copy(ys_hbm.at[p1_v.at[j]], rows1_v.at[b], sem_in),
            )

        pend_in = issue_in(0)
        pend_out = None
        for j in range(NSUBC):
            b = j % 2
            for d in pend_in:
                d.wait()
            if j + 1 < NSUBC:
                if pend_out is not None:
                    pend_out.wait()
                    pend_out = None
                pend_in = issue_in(j + 1)

            def body(r, _):
                for c in range(D // L):
                    rows0_v[b, r, pl.ds(c * L, L)] = (
                        rows0_v[b, r, pl.ds(c * L, L)]
                        + rows1_v[b, r, pl.ds(c * L, L)])
                return 0

            lax.fori_loop(0, SUBC, body, 0)
            if pend_out is not None:
                pend_out.wait()
            pend_out = pltpu.async_copy(
                rows0_v.at[b], out_hbm.at[pl.ds(tbase + j * SUBC, SUBC)],
                sem_out)
        pend_out.wait()

    out = _combine(ys, p0, p1)
    return out.reshape(Bq, Sq, D)


# speedup vs baseline: 20.2152x; 1.0386x over previous
import functools

import jax
import jax.numpy as jnp
from jax import lax
from jax.experimental import pallas as pl
from jax.experimental.pallas import tpu as pltpu
from jax.experimental.pallas import tpu_sc as plsc

K = 2
BLK = 256
GTILE = 256
WREP = 16


def _gating_tile(x_ref, wg_ref, bg_ref, topi_ref, rank_ref, g0_ref, g1_ref,
                 counts_ref, counter):
    i = pl.program_id(0)

    @pl.when(i == 0)
    def _():
        counter[...] = jnp.zeros_like(counter)

    E = wg_ref.shape[0]
    xs = x_ref[...]
    scores = lax.dot_general(
        xs, wg_ref[...], (((1,), (1,)), ((), ())),
        preferred_element_type=jnp.float32,
    ) + bg_ref[...]

    col = lax.broadcasted_iota(jnp.int32, (GTILE, E), 1)
    m1 = jnp.max(scores, axis=1, keepdims=True)
    a1 = jnp.argmax(scores, axis=1).astype(jnp.int32)
    oh1 = col == a1[:, None]
    masked = jnp.where(oh1, -jnp.inf, scores)
    m2 = jnp.max(masked, axis=1, keepdims=True)
    a2 = jnp.argmax(masked, axis=1).astype(jnp.int32)
    oh2 = col == a2[:, None]

    t = jnp.exp(m2 - m1)
    g1 = 1.0 / (1.0 + t)
    g2 = 1.0 - g1

    occ = oh1.astype(jnp.float32) + oh2.astype(jnp.float32)
    ri = lax.broadcasted_iota(jnp.int32, (GTILE, GTILE), 0)
    ci = lax.broadcasted_iota(jnp.int32, (GTILE, GTILE), 1)
    ltri = (ci < ri).astype(jnp.float32)
    cumexc = lax.dot_general(
        ltri, occ, (((1,), (0,)), ((), ())),
        preferred_element_type=jnp.float32,
    ) + counter[...]
    r0 = jnp.sum(jnp.where(oh1, cumexc, 0.0), axis=1)
    r1 = jnp.sum(jnp.where(oh2, cumexc, 0.0), axis=1)

    counter[...] = counter[...] + jnp.sum(occ, axis=0, keepdims=True)
    counts_ref[...] = counter[...]
    topi_ref[...] = jnp.concatenate([a1[:, None], a2[:, None]], axis=1)
    rank_ref[...] = jnp.concatenate(
        [r0[:, None], r1[:, None]], axis=1).astype(jnp.int32)
    g0_ref[...] = jnp.broadcast_to(g1, (GTILE, WREP))
    g1_ref[...] = jnp.broadcast_to(g2, (GTILE, WREP))


def _make_posmap(NT):
    def _posmap_tile(topi_ref, rank_ref, counts_ref, p0_ref, p1_ref, te_ref):
        E = counts_ref.shape[1]
        c = counts_ref[...]
        tiles = jnp.floor((c + (BLK - 1)) / BLK)
        ei = lax.broadcasted_iota(jnp.int32, (E, E), 0)
        ej = lax.broadcasted_iota(jnp.int32, (E, E), 1)
        cummat = (ei <= ej).astype(jnp.float32)
        tile_cum = lax.dot_general(
            tiles, cummat, (((1,), (0,)), ((), ())),
            preferred_element_type=jnp.float32,
        )
        pad_off = (tile_cum - tiles) * BLK

        col = lax.broadcasted_iota(jnp.int32, (GTILE, E), 1)
        a1 = topi_ref[:, 0][:, None]
        a2 = topi_ref[:, 1][:, None]
        o1 = jnp.sum(jnp.where(col == a1, pad_off, 0.0), axis=1)
        o2 = jnp.sum(jnp.where(col == a2, pad_off, 0.0), axis=1)
        p0_ref[...] = o1.astype(jnp.int32)[:, None] + rank_ref[:, 0][:, None]
        p1_ref[...] = o2.astype(jnp.int32)[:, None] + rank_ref[:, 1][:, None]

        @pl.when(pl.program_id(0) == 0)
        def _():
            num_active = tile_cum[0, E - 1].astype(jnp.int32)
            ids = lax.broadcasted_iota(jnp.int32, (NT, E), 0).astype(jnp.float32)
            cums = jnp.broadcast_to(tile_cum, (NT, E))
            tev = jnp.sum((cums <= ids).astype(jnp.float32), axis=1)
            tev = tev.astype(jnp.int32)
            eids = lax.broadcasted_iota(jnp.int32, (1, E), 1)
            last_e = jnp.max(
                jnp.where(c > 0, eids, -1), axis=1)[0].astype(jnp.int32)
            tid = lax.broadcasted_iota(jnp.int32, (NT,), 0)
            tev = jnp.where(tid < num_active, tev, last_e)
            te_ref[...] = jnp.concatenate(
                [tev, num_active[None]])[None, :]

    return _posmap_tile


def _ffn_tile(te_ref, xs_ref, W1_ref, b1_ref, W2_ref, b2_ref, out_ref):
    i = pl.program_id(0)
    nt = pl.num_programs(0)

    @pl.when(i < te_ref[nt])
    def _():
        xs = xs_ref[...]
        h = lax.dot_general(
            xs, W1_ref[0], (((1,), (1,)), ((), ())),
            preferred_element_type=jnp.float32,
        ) + b1_ref[0]
        h = 0.5 * h * (1.0 + lax.erf(h * 0.7071067811865476))
        out_ref[...] = lax.dot_general(
            h, W2_ref[0], (((1,), (1,)), ((), ())),
            preferred_element_type=jnp.float32,
        ) + b2_ref[0]


def kernel(x, Wg, bg, W1, b1, W2, b2):
    Bq, Sq, D = x.shape
    E, F, _ = W1.shape
    T = Bq * Sq
    A = T * K
    flat = x.reshape(T, D)
    NT = A // BLK + E
    PADDED = NT * BLK

    topi, rank, g0w, g1w, counts_f = pl.pallas_call(
        _gating_tile,
        grid=(T // GTILE,),
        in_specs=[
            pl.BlockSpec((GTILE, D), lambda i: (i, 0)),
            pl.BlockSpec((E, D), lambda i: (0, 0)),
            pl.BlockSpec((1, E), lambda i: (0, 0)),
        ],
        out_specs=[
            pl.BlockSpec((GTILE, K), lambda i: (i, 0)),
            pl.BlockSpec((GTILE, K), lambda i: (i, 0)),
            pl.BlockSpec((GTILE, WREP), lambda i: (i, 0)),
            pl.BlockSpec((GTILE, WREP), lambda i: (i, 0)),
            pl.BlockSpec((1, E), lambda i: (0, 0)),
        ],
        out_shape=[
            jax.ShapeDtypeStruct((T, K), jnp.int32),
            jax.ShapeDtypeStruct((T, K), jnp.int32),
            jax.ShapeDtypeStruct((T, WREP), jnp.float32),
            jax.ShapeDtypeStruct((T, WREP), jnp.float32),
            jax.ShapeDtypeStruct((1, E), jnp.float32),
        ],
        scratch_shapes=[pltpu.VMEM((1, E), jnp.float32)],
    )(flat, Wg, bg.reshape(1, E))

    p0, p1, te2 = pl.pallas_call(
        _make_posmap(NT),
        grid=(T // GTILE,),
        in_specs=[
            pl.BlockSpec((GTILE, K), lambda i: (i, 0)),
            pl.BlockSpec((GTILE, K), lambda i: (i, 0)),
            pl.BlockSpec((1, E), lambda i: (0, 0)),
        ],
        out_specs=[
            pl.BlockSpec((GTILE, 1), lambda i: (i, 0)),
            pl.BlockSpec((GTILE, 1), lambda i: (i, 0)),
            pl.BlockSpec((1, NT + 1), lambda i: (0, 0)),
        ],
        out_shape=[
            jax.ShapeDtypeStruct((T, 1), jnp.int32),
            jax.ShapeDtypeStruct((T, 1), jnp.int32),
            jax.ShapeDtypeStruct((1, NT + 1), jnp.int32),
        ],
    )(topi, rank, counts_f)
    p0 = p0.reshape(T)
    p1 = p1.reshape(T)
    te = te2.reshape(NT + 1)

    info = plsc.get_sparse_core_info()
    NC, NS = info.num_cores, info.num_subcores
    NW = NC * NS
    mesh = plsc.VectorSubcoreMesh(core_axis_name="c", subcore_axis_name="s")
    tok_per_w = T // NW

    SUBT = 32
    NSUB = tok_per_w // SUBT

    @functools.partial(
        pl.kernel,
        out_type=jax.ShapeDtypeStruct((PADDED, D), jnp.float32),
        mesh=mesh,
        scratch_types=[
            pltpu.VMEM((NSUB, SUBT), jnp.int32),
            pltpu.VMEM((NSUB, SUBT), jnp.int32),
            pltpu.VMEM((NSUB, SUBT), jnp.int32),
            pltpu.VMEM((2, SUBT, D), jnp.float32),
            pltpu.SemaphoreType.DMA,
            pltpu.SemaphoreType.DMA,
        ],
    )
    def _dispatch(flat_hbm, p0_hbm, p1_hbm, xs_hbm,
                  tok_v, p0_v, p1_v, rows_v, sem_in, sem_out):
        wid = lax.axis_index("s") * NC + lax.axis_index("c")
        tbase = wid * tok_per_w
        L = 16
        for j in range(NSUB):
            pltpu.sync_copy(p0_hbm.at[pl.ds(tbase + j * SUBT, SUBT)], p0_v.at[j])
            pltpu.sync_copy(p1_hbm.at[pl.ds(tbase + j * SUBT, SUBT)], p1_v.at[j])
            for v in range(SUBT // L):
                tok_v[j, pl.ds(v * L, L)] = (
                    lax.broadcasted_iota(jnp.int32, (L,), 0)
                    + (tbase + j * SUBT + v * L))

        def issue_in(j):
            b = j % 2
            return pltpu.async_copy(
                flat_hbm.at[tok_v.at[j]], rows_v.at[b], sem_in)

        def issue_out(j):
            b = j % 2
            return (
                pltpu.async_copy(rows_v.at[b], xs_hbm.at[p0_v.at[j]], sem_out),
                pltpu.async_copy(rows_v.at[b], xs_hbm.at[p1_v.at[j]], sem_out),
            )

        pend_in = issue_in(0)
        pend_out = None
        for j in range(NSUB):
            pend_in.wait()
            if j + 1 < NSUB:
                if pend_out is not None:
                    for d in pend_out:
                        d.wait()
                    pend_out = None
                pend_in = issue_in(j + 1)
            if pend_out is not None:
                for d in pend_out:
                    d.wait()
            pend_out = issue_out(j)
        for d in pend_out:
            d.wait()

    xs = _dispatch(flat, p0, p1)

    grid_spec = pltpu.PrefetchScalarGridSpec(
        num_scalar_prefetch=1,
        grid=(NT,),
        in_specs=[
            pl.BlockSpec((BLK, D), lambda i, te: (i, 0)),
            pl.BlockSpec((1, F, D), lambda i, te: (te[i], 0, 0)),
            pl.BlockSpec((1, 1, F), lambda i, te: (te[i], 0, 0)),
            pl.BlockSpec((1, D, F), lambda i, te: (te[i], 0, 0)),
            pl.BlockSpec((1, 1, D), lambda i, te: (te[i], 0, 0)),
        ],
        out_specs=pl.BlockSpec((BLK, D), lambda i, te: (i, 0)),
    )
    ys = pl.pallas_call(
        _ffn_tile,
        grid_spec=grid_spec,
        out_shape=jax.ShapeDtypeStruct((PADDED, D), jnp.float32),
    )(te, xs, W1, b1.reshape(E, 1, F), W2, b2.reshape(E, 1, D))

    SUBC = 32
    NSUBC = tok_per_w // SUBC

    @functools.partial(
        pl.kernel,
        out_type=jax.ShapeDtypeStruct((T, D), jnp.float32),
        mesh=mesh,
        scratch_types=[
            pltpu.VMEM((NSUBC, SUBC), jnp.int32),
            pltpu.VMEM((NSUBC, SUBC), jnp.int32),
            pltpu.VMEM((2, SUBC, D), jnp.float32),
            pltpu.VMEM((2, SUBC, D), jnp.float32),
            pltpu.VMEM((2, SUBC, WREP), jnp.float32),
            pltpu.VMEM((2, SUBC, WREP), jnp.float32),
            pltpu.SemaphoreType.DMA,
            pltpu.SemaphoreType.DMA,
        ],
    )
    def _combine(ys_hbm, p0_hbm, p1_hbm, g0_hbm, g1_hbm, out_hbm,
                 p0_v, p1_v, rows0_v, rows1_v, g0_v, g1_v, sem_in, sem_out):
        wid = lax.axis_index("s") * NC + lax.axis_index("c")
        tbase = wid * tok_per_w
        L = 16
        for j in range(NSUBC):
            pltpu.sync_copy(p0_hbm.at[pl.ds(tbase + j * SUBC, SUBC)], p0_v.at[j])
            pltpu.sync_copy(p1_hbm.at[pl.ds(tbase + j * SUBC, SUBC)], p1_v.at[j])

        def issue_in(j):
            b = j % 2
            return (
                pltpu.async_copy(ys_hbm.at[p0_v.at[j]], rows0_v.at[b], sem_in),
                pltpu.async_copy(ys_hbm.at[p1_v.at[j]], rows1_v.at[b], sem_in),
                pltpu.async_copy(
                    g0_hbm.at[pl.ds(tbase + j * SUBC, SUBC)], g0_v.at[b], sem_in),
                pltpu.async_copy(
                    g1_hbm.at[pl.ds(tbase + j * SUBC, SUBC)], g1_v.at[b], sem_in),
            )

        pend_in = issue_in(0)
        pend_out = None
        for j in range(NSUBC):
            b = j % 2
            for d in pend_in:
                d.wait()
            if j + 1 < NSUBC:
                if pend_out is not None:
                    pend_out.wait()
                    pend_out = None
                pend_in = issue_in(j + 1)

            def body(r, _):
                b0 = g0_v[b, r, pl.ds(0, L)]
                b1 = g1_v[b, r, pl.ds(0, L)]
                for c in range(D // L):
                    rows0_v[b, r, pl.ds(c * L, L)] = (
                        rows0_v[b, r, pl.ds(c * L, L)] * b0
                        + rows1_v[b, r, pl.ds(c * L, L)] * b1)
                return 0

            lax.fori_loop(0, SUBC, body, 0)
            if pend_out is not None:
                pend_out.wait()
            pend_out = pltpu.async_copy(
                rows0_v.at[b], out_hbm.at[pl.ds(tbase + j * SUBC, SUBC)],
                sem_out)
        pend_out.wait()

    out = _combine(ys, p0, p1, g0w, g1w)
    return out.reshape(Bq, Sq, D)


# speedup vs baseline: 21.3312x; 1.0552x over previous
import functools

import jax
import jax.numpy as jnp
from jax import lax
from jax.experimental import pallas as pl
from jax.experimental.pallas import tpu as pltpu
from jax.experimental.pallas import tpu_sc as plsc

K = 2
BLK = 256
GTILE = 256
WREP = 16


def _gating_tile(x_ref, wg_ref, bg_ref, topi_ref, rank_ref, g0_ref, g1_ref,
                 counts_ref, counter):
    i = pl.program_id(0)

    @pl.when(i == 0)
    def _():
        counter[...] = jnp.zeros_like(counter)

    E = wg_ref.shape[0]
    xs = x_ref[...]
    scores = lax.dot_general(
        xs, wg_ref[...], (((1,), (1,)), ((), ())),
        preferred_element_type=jnp.float32,
    ) + bg_ref[...]

    col = lax.broadcasted_iota(jnp.int32, (GTILE, E), 1)
    m1 = jnp.max(scores, axis=1, keepdims=True)
    a1 = jnp.argmax(scores, axis=1).astype(jnp.int32)
    oh1 = col == a1[:, None]
    masked = jnp.where(oh1, -jnp.inf, scores)
    m2 = jnp.max(masked, axis=1, keepdims=True)
    a2 = jnp.argmax(masked, axis=1).astype(jnp.int32)
    oh2 = col == a2[:, None]

    t = jnp.exp(m2 - m1)
    g1 = 1.0 / (1.0 + t)
    g2 = 1.0 - g1

    occ = oh1.astype(jnp.float32) + oh2.astype(jnp.float32)
    ri = lax.broadcasted_iota(jnp.int32, (GTILE, GTILE), 0)
    ci = lax.broadcasted_iota(jnp.int32, (GTILE, GTILE), 1)
    ltri = (ci < ri).astype(jnp.float32)
    cumexc = lax.dot_general(
        ltri, occ, (((1,), (0,)), ((), ())),
        preferred_element_type=jnp.float32,
    ) + counter[...]
    r0 = jnp.sum(jnp.where(oh1, cumexc, 0.0), axis=1)
    r1 = jnp.sum(jnp.where(oh2, cumexc, 0.0), axis=1)

    counter[...] = counter[...] + jnp.sum(occ, axis=0, keepdims=True)
    counts_ref[...] = counter[...]
    topi_ref[...] = jnp.concatenate([a1[:, None], a2[:, None]], axis=1)
    rank_ref[...] = jnp.concatenate(
        [r0[:, None], r1[:, None]], axis=1).astype(jnp.int32)
    g0_ref[...] = jnp.broadcast_to(g1, (GTILE, WREP))
    g1_ref[...] = jnp.broadcast_to(g2, (GTILE, WREP))


def _make_posmap(NT):
    def _posmap_tile(topi_ref, rank_ref, counts_ref, p0_ref, p1_ref, te_ref):
        E = counts_ref.shape[1]
        c = counts_ref[...]
        tiles = jnp.floor((c + (BLK - 1)) / BLK)
        ei = lax.broadcasted_iota(jnp.int32, (E, E), 0)
        ej = lax.broadcasted_iota(jnp.int32, (E, E), 1)
        cummat = (ei <= ej).astype(jnp.float32)
        tile_cum = lax.dot_general(
            tiles, cummat, (((1,), (0,)), ((), ())),
            preferred_element_type=jnp.float32,
        )
        pad_off = (tile_cum - tiles) * BLK

        col = lax.broadcasted_iota(jnp.int32, (GTILE, E), 1)
        a1 = topi_ref[:, 0][:, None]
        a2 = topi_ref[:, 1][:, None]
        o1 = jnp.sum(jnp.where(col == a1, pad_off, 0.0), axis=1)
        o2 = jnp.sum(jnp.where(col == a2, pad_off, 0.0), axis=1)
        p0_ref[...] = o1.astype(jnp.int32)[:, None] + rank_ref[:, 0][:, None]
        p1_ref[...] = o2.astype(jnp.int32)[:, None] + rank_ref[:, 1][:, None]

        @pl.when(pl.program_id(0) == 0)
        def _():
            num_active = tile_cum[0, E - 1].astype(jnp.int32)
            ids = lax.broadcasted_iota(jnp.int32, (NT, E), 0).astype(jnp.float32)
            cums = jnp.broadcast_to(tile_cum, (NT, E))
            tev = jnp.sum((cums <= ids).astype(jnp.float32), axis=1)
            tev = tev.astype(jnp.int32)
            eids = lax.broadcasted_iota(jnp.int32, (1, E), 1)
            last_e = jnp.max(
                jnp.where(c > 0, eids, -1), axis=1)[0].astype(jnp.int32)
            tid = lax.broadcasted_iota(jnp.int32, (NT,), 0)
            tev = jnp.where(tid < num_active, tev, last_e)
            te_ref[...] = jnp.concatenate(
                [tev, num_active[None]])[None, :]

    return _posmap_tile


def _ffn_tile(te_ref, xs_ref, W1_ref, b1_ref, W2_ref, b2_ref, out_ref):
    i = pl.program_id(0)
    nt = pl.num_programs(0)

    @pl.when(i < te_ref[nt])
    def _():
        xs = xs_ref[...]
        h = lax.dot_general(
            xs, W1_ref[0], (((1,), (1,)), ((), ())),
            preferred_element_type=jnp.float32,
        ) + b1_ref[0]
        h = 0.5 * h * (1.0 + lax.erf(h * 0.7071067811865476))
        out_ref[...] = lax.dot_general(
            h, W2_ref[0], (((1,), (1,)), ((), ())),
            preferred_element_type=jnp.float32,
        ) + b2_ref[0]


def kernel(x, Wg, bg, W1, b1, W2, b2):
    Bq, Sq, D = x.shape
    E, F, _ = W1.shape
    T = Bq * Sq
    A = T * K
    flat = x.reshape(T, D)
    NT = A // BLK + E
    PADDED = NT * BLK

    topi, rank, g0w, g1w, counts_f = pl.pallas_call(
        _gating_tile,
        grid=(T // GTILE,),
        in_specs=[
            pl.BlockSpec((GTILE, D), lambda i: (i, 0)),
            pl.BlockSpec((E, D), lambda i: (0, 0)),
            pl.BlockSpec((1, E), lambda i: (0, 0)),
        ],
        out_specs=[
            pl.BlockSpec((GTILE, K), lambda i: (i, 0)),
            pl.BlockSpec((GTILE, K), lambda i: (i, 0)),
            pl.BlockSpec((GTILE, WREP), lambda i: (i, 0)),
            pl.BlockSpec((GTILE, WREP), lambda i: (i, 0)),
            pl.BlockSpec((1, E), lambda i: (0, 0)),
        ],
        out_shape=[
            jax.ShapeDtypeStruct((T, K), jnp.int32),
            jax.ShapeDtypeStruct((T, K), jnp.int32),
            jax.ShapeDtypeStruct((T, WREP), jnp.float32),
            jax.ShapeDtypeStruct((T, WREP), jnp.float32),
            jax.ShapeDtypeStruct((1, E), jnp.float32),
        ],
        scratch_shapes=[pltpu.VMEM((1, E), jnp.float32)],
    )(flat, Wg, bg.reshape(1, E))

    p0, p1, te2 = pl.pallas_call(
        _make_posmap(NT),
        grid=(T // GTILE,),
        in_specs=[
            pl.BlockSpec((GTILE, K), lambda i: (i, 0)),
            pl.BlockSpec((GTILE, K), lambda i: (i, 0)),
            pl.BlockSpec((1, E), lambda i: (0, 0)),
        ],
        out_specs=[
            pl.BlockSpec((GTILE, 1), lambda i: (i, 0)),
            pl.BlockSpec((GTILE, 1), lambda i: (i, 0)),
            pl.BlockSpec((1, NT + 1), lambda i: (0, 0)),
        ],
        out_shape=[
            jax.ShapeDtypeStruct((T, 1), jnp.int32),
            jax.ShapeDtypeStruct((T, 1), jnp.int32),
            jax.ShapeDtypeStruct((1, NT + 1), jnp.int32),
        ],
    )(topi, rank, counts_f)
    p0 = p0.reshape(T)
    p1 = p1.reshape(T)
    te = te2.reshape(NT + 1)

    info = plsc.get_sparse_core_info()
    NC, NS = info.num_cores, info.num_subcores
    NW = NC * NS
    mesh = plsc.VectorSubcoreMesh(core_axis_name="c", subcore_axis_name="s")
    tok_per_w = T // NW

    SUBT = 64
    NSUB = tok_per_w // SUBT

    @functools.partial(
        pl.kernel,
        out_type=jax.ShapeDtypeStruct((PADDED, D), jnp.float32),
        mesh=mesh,
        scratch_types=[
            pltpu.VMEM((NSUB, SUBT), jnp.int32),
            pltpu.VMEM((NSUB, SUBT), jnp.int32),
            pltpu.VMEM((NSUB, SUBT), jnp.int32),
            pltpu.VMEM((2, SUBT, D), jnp.float32),
            pltpu.SemaphoreType.DMA,
            pltpu.SemaphoreType.DMA,
        ],
    )
    def _dispatch(flat_hbm, p0_hbm, p1_hbm, xs_hbm,
                  tok_v, p0_v, p1_v, rows_v, sem_in, sem_out):
        wid = lax.axis_index("s") * NC + lax.axis_index("c")
        tbase = wid * tok_per_w
        L = 16
        for j in range(NSUB):
            pltpu.sync_copy(p0_hbm.at[pl.ds(tbase + j * SUBT, SUBT)], p0_v.at[j])
            pltpu.sync_copy(p1_hbm.at[pl.ds(tbase + j * SUBT, SUBT)], p1_v.at[j])
            for v in range(SUBT // L):
                tok_v[j, pl.ds(v * L, L)] = (
                    lax.broadcasted_iota(jnp.int32, (L,), 0)
                    + (tbase + j * SUBT + v * L))

        def issue_in(j):
            b = j % 2
            return pltpu.async_copy(
                flat_hbm.at[tok_v.at[j]], rows_v.at[b], sem_in)

        def issue_out(j):
            b = j % 2
            return (
                pltpu.async_copy(rows_v.at[b], xs_hbm.at[p0_v.at[j]], sem_out),
                pltpu.async_copy(rows_v.at[b], xs_hbm.at[p1_v.at[j]], sem_out),
            )

        pend_in = issue_in(0)
        pend_out = None
        for j in range(NSUB):
            pend_in.wait()
            if j + 1 < NSUB:
                if pend_out is not None:
                    for d in pend_out:
                        d.wait()
                    pend_out = None
                pend_in = issue_in(j + 1)
            if pend_out is not None:
                for d in pend_out:
                    d.wait()
            pend_out = issue_out(j)
        for d in pend_out:
            d.wait()

    xs = _dispatch(flat, p0, p1)

    def _row_ix(i, te):
        return (jnp.minimum(i, te[NT] - 1), 0)

    grid_spec = pltpu.PrefetchScalarGridSpec(
        num_scalar_prefetch=1,
        grid=(NT,),
        in_specs=[
            pl.BlockSpec((BLK, D), _row_ix),
            pl.BlockSpec((1, F, D), lambda i, te: (te[i], 0, 0)),
            pl.BlockSpec((1, 1, F), lambda i, te: (te[i], 0, 0)),
            pl.BlockSpec((1, D, F), lambda i, te: (te[i], 0, 0)),
            pl.BlockSpec((1, 1, D), lambda i, te: (te[i], 0, 0)),
        ],
        out_specs=pl.BlockSpec((BLK, D), _row_ix),
    )
    ys = pl.pallas_call(
        _ffn_tile,
        grid_spec=grid_spec,
        out_shape=jax.ShapeDtypeStruct((PADDED, D), jnp.float32),
    )(te, xs, W1, b1.reshape(E, 1, F), W2, b2.reshape(E, 1, D))

    SUBC = 32
    NSUBC = tok_per_w // SUBC

    @functools.partial(
        pl.kernel,
        out_type=jax.ShapeDtypeStruct((T, D), jnp.float32),
        mesh=mesh,
        scratch_types=[
            pltpu.VMEM((NSUBC, SUBC), jnp.int32),
            pltpu.VMEM((NSUBC, SUBC), jnp.int32),
            pltpu.VMEM((2, SUBC, D), jnp.float32),
            pltpu.VMEM((2, SUBC, D), jnp.float32),
            pltpu.VMEM((2, SUBC, WREP), jnp.float32),
            pltpu.VMEM((2, SUBC, WREP), jnp.float32),
            pltpu.SemaphoreType.DMA,
            pltpu.SemaphoreType.DMA,
        ],
    )
    def _combine(ys_hbm, p0_hbm, p1_hbm, g0_hbm, g1_hbm, out_hbm,
                 p0_v, p1_v, rows0_v, rows1_v, g0_v, g1_v, sem_in, sem_out):
        wid = lax.axis_index("s") * NC + lax.axis_index("c")
        tbase = wid * tok_per_w
        L = 16
        for j in range(NSUBC):
            pltpu.sync_copy(p0_hbm.at[pl.ds(tbase + j * SUBC, SUBC)], p0_v.at[j])
            pltpu.sync_copy(p1_hbm.at[pl.ds(tbase + j * SUBC, SUBC)], p1_v.at[j])

        def issue_in(j):
            b = j % 2
            return (
                pltpu.async_copy(ys_hbm.at[p0_v.at[j]], rows0_v.at[b], sem_in),
                pltpu.async_copy(ys_hbm.at[p1_v.at[j]], rows1_v.at[b], sem_in),
                pltpu.async_copy(
                    g0_hbm.at[pl.ds(tbase + j * SUBC, SUBC)], g0_v.at[b], sem_in),
                pltpu.async_copy(
                    g1_hbm.at[pl.ds(tbase + j * SUBC, SUBC)], g1_v.at[b], sem_in),
            )

        pend_in = issue_in(0)
        pend_out = None
        for j in range(NSUBC):
            b = j % 2
            for d in pend_in:
                d.wait()
            if j + 1 < NSUBC:
                if pend_out is not None:
                    pend_out.wait()
                    pend_out = None
                pend_in = issue_in(j + 1)

            def body(r, _):
                b0 = g0_v[b, r, pl.ds(0, L)]
                b1 = g1_v[b, r, pl.ds(0, L)]
                for c in range(D // L):
                    rows0_v[b, r, pl.ds(c * L, L)] = (
                        rows0_v[b, r, pl.ds(c * L, L)] * b0
                        + rows1_v[b, r, pl.ds(c * L, L)] * b1)
                return 0

            lax.fori_loop(0, SUBC, body, 0)
            if pend_out is not None:
                pend_out.wait()
            pend_out = pltpu.async_copy(
                rows0_v.at[b], out_hbm.at[pl.ds(tbase + j * SUBC, SUBC)],
                sem_out)
        pend_out.wait()

    out = _combine(ys, p0, p1, g0w, g1w)
    return out.reshape(Bq, Sq, D)
